# baseline (speedup 1.0000x reference)
import functools

import jax
import jax.numpy as jnp
from jax import lax
from jax.experimental import pallas as pl
from jax.experimental.pallas import tpu as pltpu
from jax.experimental.pallas import tpu_sc as plsc

_NC = 2
_NS = 16
_L = 16
_CB = 80


def _proj_body(h_ref, ws_ref, wd_ref, ps_ref, pd_ref):
    hb = h_ref[...]
    ps = jnp.dot(hb, ws_ref[...], preferred_element_type=jnp.float32)
    pd = jnp.dot(hb, wd_ref[...], preferred_element_type=jnp.float32)
    ps_ref[...] = ps.astype(jnp.bfloat16)
    pd_ref[...] = pd.astype(jnp.bfloat16)


def _project(h, w_src, w_dst):
    n, d = h.shape
    bn = 2000
    return pl.pallas_call(
        _proj_body,
        grid=(n // bn,),
        in_specs=[
            pl.BlockSpec((bn, d), lambda i: (i, 0)),
            pl.BlockSpec((d, 2 * d), lambda i: (0, 0)),
            pl.BlockSpec((d, 2 * d), lambda i: (0, 0)),
        ],
        out_specs=[
            pl.BlockSpec((bn, 2 * d), lambda i: (i, 0)),
            pl.BlockSpec((bn, 2 * d), lambda i: (i, 0)),
        ],
        out_shape=[
            jax.ShapeDtypeStruct((n, 2 * d), jnp.bfloat16),
            jax.ShapeDtypeStruct((n, 2 * d), jnp.bfloat16),
        ],
    )(h, w_src, w_dst)


def _gather_body(e, d2, ps_hbm, pd_hbm, src_hbm, dst_hbm, g_hbm,
                 idx_s, idx_d, buf_a, buf_b, sem_a, sem_b):
    c = lax.axis_index("c")
    s = lax.axis_index("s")
    wid = c * _NS + s
    ept = e // (_NC * _NS)
    base = wid * ept
    n_chunks = ept // _CB

    def chunk(ci, carry):
        off = base + ci * _CB
        pltpu.sync_copy(src_hbm.at[pl.ds(off, _CB)], idx_s)
        pltpu.sync_copy(dst_hbm.at[pl.ds(off, _CB)], idx_d)
        cp_a = pltpu.async_copy(ps_hbm.at[idx_s], buf_a, sem_a)
        cp_b = pltpu.async_copy(pd_hbm.at[idx_d], buf_b, sem_b)
        cp_a.wait()
        cp_b.wait()

        def add_row(r, cc):
            for k in range(d2 // (2 * _L)):
                sl = pl.ds(k * 2 * _L, 2 * _L)
                buf_a[r, sl] = buf_a[r, sl] + buf_b[r, sl]
            return cc

        lax.fori_loop(0, _CB, add_row, 0)
        pltpu.sync_copy(buf_a, g_hbm.at[pl.ds(off, _CB)])
        return carry

    lax.fori_loop(0, n_chunks, chunk, 0)


def _gather_add(p_src, p_dst, src, dst):
    n, d2 = p_src.shape
    e = src.shape[0]
    mesh = plsc.VectorSubcoreMesh(core_axis_name="c", subcore_axis_name="s")
    kfn = pl.kernel(
        functools.partial(_gather_body, e, d2),
        out_type=jax.ShapeDtypeStruct((e, d2), jnp.bfloat16),
        mesh=mesh,
        compiler_params=pltpu.CompilerParams(use_tc_tiling_on_sc=False),
        scratch_types=[
            pltpu.VMEM((_CB,), jnp.int32),
            pltpu.VMEM((_CB,), jnp.int32),
            pltpu.VMEM((_CB, d2), jnp.bfloat16),
            pltpu.VMEM((_CB, d2), jnp.bfloat16),
            pltpu.SemaphoreType.DMA,
            pltpu.SemaphoreType.DMA,
        ],
    )
    return kfn(p_src, p_dst, src, dst)


def _edge_body(d, g_ref, ea_ref, we_ref, b_ref, out_ref):
    x = (g_ref[...].astype(jnp.float32)
         + jnp.dot(ea_ref[...], we_ref[...], preferred_element_type=jnp.float32)
         + b_ref[...])
    xg = x[:, :d]
    xm = x[:, d:]
    gate = 1.0 / (1.0 + jnp.exp(-xg))
    msg = jnp.maximum(xm, 0.0) + jnp.log(1.0 + jnp.exp(-jnp.abs(xm)))
    out_ref[...] = gate * msg


def _edge_mlp(g, edge_attr, w_edge, b_cat):
    e, d2 = g.shape
    de = edge_attr.shape[1]
    d = d2 // 2
    be = 2000
    return pl.pallas_call(
        functools.partial(_edge_body, d),
        grid=(e // be,),
        in_specs=[
            pl.BlockSpec((be, d2), lambda i: (i, 0)),
            pl.BlockSpec((be, de), lambda i: (i, 0)),
            pl.BlockSpec((de, d2), lambda i: (0, 0)),
            pl.BlockSpec((1, d2), lambda i: (0, 0)),
        ],
        out_specs=pl.BlockSpec((be, d), lambda i: (i, 0)),
        out_shape=jax.ShapeDtypeStruct((e, d), jnp.float32),
    )(g, edge_attr, w_edge, b_cat)


_RB = 128


def _scatter_body(e, n_pad, d, gm_hbm, dst_hbm, s_out, c_out,
                  idx_v, gm_buf, ones_buf, row_buf, cnt_buf, s_sh, c_sh):
    c = lax.axis_index("c")
    s = lax.axis_index("s")
    wid = c * _NS + s
    ept = e // (_NC * _NS)
    base = wid * ept
    rpt = n_pad // _NS
    r0 = s * rpt

    def fill(j, cc):
        ones_buf[j, :] = jnp.full((_L,), 1.0, jnp.float32)
        return cc

    lax.fori_loop(0, _CB, fill, 0)

    zv = jnp.zeros((_L,), jnp.float32)

    def zrow(j, cc):
        for k in range(d // _L):
            row_buf[j, pl.ds(k * _L, _L)] = zv
        return cc

    lax.fori_loop(0, _RB, zrow, 0)

    def zcnt(j, cc):
        cnt_buf[j, :] = zv
        return cc

    lax.fori_loop(0, rpt, zcnt, 0)
    for j in range(rpt // _RB):
        pltpu.sync_copy(row_buf, s_sh.at[pl.ds(r0 + j * _RB, _RB)])
    pltpu.sync_copy(cnt_buf, c_sh.at[pl.ds(r0, rpt)])
    plsc.subcore_barrier()

    def chunk(ci, carry):
        off = base + ci * _CB
        pltpu.sync_copy(dst_hbm.at[pl.ds(off, _CB)], idx_v)
        pltpu.sync_copy(gm_hbm.at[pl.ds(off, _CB)], gm_buf)
        pltpu.sync_copy(gm_buf, s_sh.at[idx_v], add=True)
        pltpu.sync_copy(ones_buf, c_sh.at[idx_v], add=True)
        return carry

    lax.fori_loop(0, ept // _CB, chunk, 0)
    plsc.subcore_barrier()
    for j in range(rpt // _RB):
        pltpu.sync_copy(s_sh.at[pl.ds(r0 + j * _RB, _RB)], row_buf)
        pltpu.sync_copy(row_buf, s_out.at[c, pl.ds(r0 + j * _RB, _RB)])
    pltpu.sync_copy(c_sh.at[pl.ds(r0, rpt)], cnt_buf)
    pltpu.sync_copy(cnt_buf, c_out.at[c, pl.ds(r0, rpt)])


def _scatter_mean_parts(gm, dst, n):
    e, d = gm.shape
    blk = _NS * _RB
    n_pad = ((n + blk - 1) // blk) * blk
    mesh = plsc.VectorSubcoreMesh(core_axis_name="c", subcore_axis_name="s")
    kfn = pl.kernel(
        functools.partial(_scatter_body, e, n_pad, d),
        out_type=[
            jax.ShapeDtypeStruct((_NC, n_pad, d), jnp.float32),
            jax.ShapeDtypeStruct((_NC, n_pad, _L), jnp.float32),
        ],
        mesh=mesh,
        compiler_params=pltpu.CompilerParams(use_tc_tiling_on_sc=False),
        scratch_types=[
            pltpu.VMEM((_CB,), jnp.int32),
            pltpu.VMEM((_CB, d), jnp.float32),
            pltpu.VMEM((_CB, _L), jnp.float32),
            pltpu.VMEM((_RB, d), jnp.float32),
            pltpu.VMEM((n_pad // _NS, _L), jnp.float32),
            pltpu.VMEM_SHARED((n_pad, d), jnp.float32),
            pltpu.VMEM_SHARED((n_pad, _L), jnp.float32),
        ],
    )
    return kfn(gm, dst)


def _fin_body(h_ref, s_ref, c_ref, out_ref):
    ssum = s_ref[0] + s_ref[1]
    cnt = c_ref[0, :, 0:1] + c_ref[1, :, 0:1]
    out_ref[...] = h_ref[...] + ssum / jnp.maximum(cnt, 1.0)


def _finalize(h, s_parts, c_parts):
    n, d = h.shape
    bn = 2000
    return pl.pallas_call(
        _fin_body,
        grid=(n // bn,),
        in_specs=[
            pl.BlockSpec((bn, d), lambda i: (i, 0)),
            pl.BlockSpec((_NC, bn, d), lambda i: (0, i, 0)),
            pl.BlockSpec((_NC, bn, _L), lambda i: (0, i, 0)),
        ],
        out_specs=pl.BlockSpec((bn, d), lambda i: (i, 0)),
        out_shape=jax.ShapeDtypeStruct((n, d), jnp.float32),
    )(h, s_parts, c_parts)


def kernel(h, edge_index, edge_attr, W_e, b_e, W_n, b_n):
    n, d = h.shape
    src = edge_index[0]
    dst = edge_index[1]
    w_src = jnp.concatenate([W_e[:d], W_n[:d]], axis=1)
    w_dst = jnp.concatenate([W_e[d:2 * d], W_n[d:2 * d]], axis=1)
    w_edge = jnp.concatenate([W_e[2 * d:], W_n[2 * d:]], axis=1)
    b_cat = jnp.concatenate([b_e, b_n])[None, :]

    p_src, p_dst = _project(h, w_src, w_dst)
    g = _gather_add(p_src, p_dst, src, dst)
    gm = _edge_mlp(g, edge_attr, w_edge, b_cat)
    s_parts, c_parts = _scatter_mean_parts(gm, dst, n)
    return _finalize(h, s_parts, c_parts)


# speedup vs baseline: 1.3247x; 1.3247x over previous
import functools

import jax
import jax.numpy as jnp
from jax import lax
from jax.experimental import pallas as pl
from jax.experimental.pallas import tpu as pltpu
from jax.experimental.pallas import tpu_sc as plsc

_NC = 2
_NS = 16
_L = 16
_CB = 80


def _proj_body(h_ref, ws_ref, wd_ref, ps_ref, pd_ref):
    hb = h_ref[...]
    ps_ref[...] = jnp.dot(hb, ws_ref[...], preferred_element_type=jnp.float32)
    pd_ref[...] = jnp.dot(hb, wd_ref[...], preferred_element_type=jnp.float32)


def _project(h, w_src, w_dst):
    n, d = h.shape
    bn = 2000
    return pl.pallas_call(
        _proj_body,
        grid=(n // bn,),
        in_specs=[
            pl.BlockSpec((bn, d), lambda i: (i, 0)),
            pl.BlockSpec((d, 2 * d), lambda i: (0, 0)),
            pl.BlockSpec((d, 2 * d), lambda i: (0, 0)),
        ],
        out_specs=[
            pl.BlockSpec((bn, 2 * d), lambda i: (i, 0)),
            pl.BlockSpec((bn, 2 * d), lambda i: (i, 0)),
        ],
        out_shape=[
            jax.ShapeDtypeStruct((n, 2 * d), jnp.float32),
            jax.ShapeDtypeStruct((n, 2 * d), jnp.float32),
        ],
    )(h, w_src, w_dst)


def _gather_body(e, d2, ps_hbm, pd_hbm, src3_hbm, dst3_hbm, g_hbm,
                 idx_s, idx_d, buf_a, buf_b, sem_a, sem_b):
    c = lax.axis_index("c")
    s = lax.axis_index("s")
    wid = c * _NS + s
    ept = e // (_NC * _NS)
    base = wid * ept
    n_chunks = ept // _CB

    pltpu.sync_copy(src3_hbm.at[wid], idx_s)
    pltpu.sync_copy(dst3_hbm.at[wid], idx_d)

    def chunk(ci, carry):
        off = base + ci * _CB
        cp_a = pltpu.async_copy(ps_hbm.at[idx_s.at[ci]], buf_a, sem_a)
        cp_b = pltpu.async_copy(pd_hbm.at[idx_d.at[ci]], buf_b, sem_b)
        cp_a.wait()
        cp_b.wait()

        def add_row(r, cc):
            for k in range(d2 // _L):
                sl = pl.ds(k * _L, _L)
                buf_a[r, sl] = buf_a[r, sl] + buf_b[r, sl]
            return cc

        lax.fori_loop(0, _CB, add_row, 0)
        pltpu.sync_copy(buf_a, g_hbm.at[pl.ds(off, _CB)])
        return carry

    lax.fori_loop(0, n_chunks, chunk, 0)


def _gather_add(p_src, p_dst, src3, dst3):
    n, d2 = p_src.shape
    nw, nch, cb = src3.shape
    e = nw * nch * cb
    mesh = plsc.VectorSubcoreMesh(core_axis_name="c", subcore_axis_name="s")
    kfn = pl.kernel(
        functools.partial(_gather_body, e, d2),
        out_type=jax.ShapeDtypeStruct((e, d2), jnp.float32),
        mesh=mesh,
        scratch_types=[
            pltpu.VMEM((nch, cb), jnp.int32),
            pltpu.VMEM((nch, cb), jnp.int32),
            pltpu.VMEM((_CB, d2), jnp.float32),
            pltpu.VMEM((_CB, d2), jnp.float32),
            pltpu.SemaphoreType.DMA,
            pltpu.SemaphoreType.DMA,
        ],
    )
    return kfn(p_src, p_dst, src3, dst3)


def _edge_body(d, g_ref, ea_ref, we_ref, b_ref, out_ref):
    x = (g_ref[...]
         + jnp.dot(ea_ref[...], we_ref[...], preferred_element_type=jnp.float32)
         + b_ref[...])
    xg = x[:, :d]
    xm = x[:, d:]
    gate = 1.0 / (1.0 + jnp.exp(-xg))
    msg = jnp.maximum(xm, 0.0) + jnp.log(1.0 + jnp.exp(-jnp.abs(xm)))
    out_ref[...] = gate * msg


def _edge_mlp(g, edge_attr, w_edge, b_cat):
    e, d2 = g.shape
    de = edge_attr.shape[1]
    d = d2 // 2
    be = 2000
    return pl.pallas_call(
        functools.partial(_edge_body, d),
        grid=(e // be,),
        in_specs=[
            pl.BlockSpec((be, d2), lambda i: (i, 0)),
            pl.BlockSpec((be, de), lambda i: (i, 0)),
            pl.BlockSpec((de, d2), lambda i: (0, 0)),
            pl.BlockSpec((1, d2), lambda i: (0, 0)),
        ],
        out_specs=pl.BlockSpec((be, d), lambda i: (i, 0)),
        out_shape=jax.ShapeDtypeStruct((e, d), jnp.float32),
    )(g, edge_attr, w_edge, b_cat)


_RB = 128


def _scatter_body(e, n_pad, d, gm_hbm, dst3_hbm, s_out,
                  idx2, gm_buf, row_buf, s_sh):
    c = lax.axis_index("c")
    s = lax.axis_index("s")
    wid = c * _NS + s
    ept = e // (_NC * _NS)
    base = wid * ept
    rpt = n_pad // _NS
    r0 = s * rpt

    zv = jnp.zeros((_L,), jnp.float32)

    def zrow(j, cc):
        for k in range(d // _L):
            row_buf[j, pl.ds(k * _L, _L)] = zv
        return cc

    lax.fori_loop(0, _RB, zrow, 0)
    pltpu.sync_copy(dst3_hbm.at[wid], idx2)
    for j in range(rpt // _RB):
        pltpu.sync_copy(row_buf, s_sh.at[pl.ds(r0 + j * _RB, _RB)])
    plsc.subcore_barrier()

    def chunk(ci, carry):
        off = base + ci * _CB
        pltpu.sync_copy(gm_hbm.at[pl.ds(off, _CB)], gm_buf)
        pltpu.sync_copy(gm_buf, s_sh.at[idx2.at[ci]], add=True)
        return carry

    lax.fori_loop(0, ept // _CB, chunk, 0)
    plsc.subcore_barrier()
    for j in range(rpt // _RB):
        pltpu.sync_copy(s_sh.at[pl.ds(r0 + j * _RB, _RB)], row_buf)
        pltpu.sync_copy(row_buf, s_out.at[c, pl.ds(r0 + j * _RB, _RB)])


def _count_body(e, n_pad, dst_hbm, c_out, idx1d, cnt1d):
    c = lax.axis_index("c")
    s = lax.axis_index("s")
    wid = c * _NS + s
    ept = e // (_NC * _NS)
    base = wid * ept

    zv = jnp.zeros((_L,), jnp.float32)
    ones_v = jnp.full((_L,), 1.0, jnp.float32)

    def zcnt(j, cc):
        cnt1d[pl.ds(j * _L, _L)] = zv
        return cc

    lax.fori_loop(0, n_pad // _L, zcnt, 0)
    pltpu.sync_copy(dst_hbm.at[pl.ds(base, ept)], idx1d)

    def step(k, cc):
        iv = idx1d[pl.ds(k * _L, _L)]
        plsc.addupdate_scatter(cnt1d, [iv], ones_v)
        return cc

    lax.fori_loop(0, ept // _L, step, 0)
    pltpu.sync_copy(cnt1d, c_out.at[wid, 0])


def _scatter_mean_parts(gm, dst, dst3, n):
    e, d = gm.shape
    blk = _NS * _RB
    n_pad = ((n + blk - 1) // blk) * blk
    nch = dst3.shape[1]
    mesh = plsc.VectorSubcoreMesh(core_axis_name="c", subcore_axis_name="s")
    kfn = pl.kernel(
        functools.partial(_scatter_body, e, n_pad, d),
        out_type=jax.ShapeDtypeStruct((_NC, n_pad, d), jnp.float32),
        mesh=mesh,
        scratch_types=[
            pltpu.VMEM((nch, _CB), jnp.int32),
            pltpu.VMEM((_CB, d), jnp.float32),
            pltpu.VMEM((_RB, d), jnp.float32),
            pltpu.VMEM_SHARED((n_pad, d), jnp.float32),
        ],
    )
    s_parts = kfn(gm, dst3)
    cfn = pl.kernel(
        functools.partial(_count_body, e, n_pad),
        out_type=jax.ShapeDtypeStruct((_NC * _NS, 1, n_pad), jnp.float32),
        mesh=plsc.VectorSubcoreMesh(core_axis_name="c", subcore_axis_name="s"),
        compiler_params=pltpu.CompilerParams(
            use_tc_tiling_on_sc=False, needs_layout_passes=False),
        scratch_types=[
            pltpu.VMEM((e // (_NC * _NS),), jnp.int32),
            pltpu.VMEM((n_pad,), jnp.float32),
        ],
    )
    c_parts = cfn(dst)
    return s_parts, c_parts


def _fin_body(bn, h_ref, s_ref, c_ref, out_ref):
    i = pl.program_id(0)
    ssum = s_ref[0] + s_ref[1]
    cblk = c_ref[:, :, pl.ds(i * bn, bn)]
    cnt = jnp.sum(cblk, axis=(0, 1))[:, None]
    out_ref[...] = h_ref[...] + ssum / jnp.maximum(cnt, 1.0)


def _finalize(h, s_parts, c_parts):
    n, d = h.shape
    n_pad = c_parts.shape[2]
    h_pad = jnp.pad(h, ((0, n_pad - n), (0, 0)))
    bn = 2048
    out = pl.pallas_call(
        functools.partial(_fin_body, bn),
        grid=(n_pad // bn,),
        in_specs=[
            pl.BlockSpec((bn, d), lambda i: (i, 0)),
            pl.BlockSpec((_NC, bn, d), lambda i: (0, i, 0)),
            pl.BlockSpec((_NC * _NS, 1, n_pad), lambda i: (0, 0, 0)),
        ],
        out_specs=pl.BlockSpec((bn, d), lambda i: (i, 0)),
        out_shape=jax.ShapeDtypeStruct((n_pad, d), jnp.float32),
    )(h_pad, s_parts, c_parts)
    return out[:n]


def kernel(h, edge_index, edge_attr, W_e, b_e, W_n, b_n):
    n, d = h.shape
    e = edge_index.shape[1]
    nw = _NC * _NS
    src = edge_index[0]
    dst = edge_index[1]
    src3 = src.reshape(nw, e // (nw * _CB), _CB)
    dst3 = dst.reshape(nw, e // (nw * _CB), _CB)
    w_src = jnp.concatenate([W_e[:d], W_n[:d]], axis=1)
    w_dst = jnp.concatenate([W_e[d:2 * d], W_n[d:2 * d]], axis=1)
    w_edge = jnp.concatenate([W_e[2 * d:], W_n[2 * d:]], axis=1)
    b_cat = jnp.concatenate([b_e, b_n])[None, :]

    p_src, p_dst = _project(h, w_src, w_dst)
    g = _gather_add(p_src, p_dst, src3, dst3)
    gm = _edge_mlp(g, edge_attr, w_edge, b_cat)
    s_parts, c_parts = _scatter_mean_parts(gm, dst, dst3, n)
    return _finalize(h, s_parts, c_parts)


# speedup vs baseline: 1.6104x; 1.2157x over previous
import functools

import jax
import jax.numpy as jnp
from jax import lax
from jax.experimental import pallas as pl
from jax.experimental.pallas import tpu as pltpu
from jax.experimental.pallas import tpu_sc as plsc

_NC = 2
_NS = 16
_L = 16
_CB = 80


def _proj_body(h_ref, ws_ref, wd_ref, ps_ref, pd_ref):
    hb = h_ref[...]
    ps_ref[...] = jnp.dot(hb, ws_ref[...], preferred_element_type=jnp.float32)
    pd_ref[...] = jnp.dot(hb, wd_ref[...], preferred_element_type=jnp.float32)


def _project(h, w_src, w_dst):
    n, d = h.shape
    bn = 2000
    return pl.pallas_call(
        _proj_body,
        grid=(n // bn,),
        in_specs=[
            pl.BlockSpec((bn, d), lambda i: (i, 0)),
            pl.BlockSpec((d, 2 * d), lambda i: (0, 0)),
            pl.BlockSpec((d, 2 * d), lambda i: (0, 0)),
        ],
        out_specs=[
            pl.BlockSpec((bn, 2 * d), lambda i: (i, 0)),
            pl.BlockSpec((bn, 2 * d), lambda i: (i, 0)),
        ],
        out_shape=[
            jax.ShapeDtypeStruct((n, 2 * d), jnp.float32),
            jax.ShapeDtypeStruct((n, 2 * d), jnp.float32),
        ],
    )(h, w_src, w_dst)


def _gather_body(e, d2, ps_hbm, pd_hbm, src3_hbm, dst3_hbm, g_hbm,
                 idx_s, idx_d, buf_a0, buf_b0, buf_a1, buf_b1,
                 sem_a0, sem_b0, sem_a1, sem_b1):
    c = lax.axis_index("c")
    s = lax.axis_index("s")
    wid = c * _NS + s
    ept = e // (_NC * _NS)
    base = wid * ept
    n_chunks = ept // _CB

    pltpu.sync_copy(src3_hbm.at[wid], idx_s)
    pltpu.sync_copy(dst3_hbm.at[wid], idx_d)

    def start(ci, ba, bb, sa, sb):
        pltpu.async_copy(ps_hbm.at[idx_s.at[ci]], ba, sa)
        pltpu.async_copy(pd_hbm.at[idx_d.at[ci]], bb, sb)

    def finish(ci, ba, bb, sa, sb):
        pltpu.make_async_copy(ps_hbm.at[idx_s.at[ci]], ba, sa).wait()
        pltpu.make_async_copy(pd_hbm.at[idx_d.at[ci]], bb, sb).wait()

        def add_row(r, cc):
            for k in range(d2 // _L):
                sl = pl.ds(k * _L, _L)
                ba[r, sl] = ba[r, sl] + bb[r, sl]
            return cc

        lax.fori_loop(0, _CB, add_row, 0)
        pltpu.sync_copy(ba, g_hbm.at[pl.ds(base + ci * _CB, _CB)])

    start(0, buf_a0, buf_b0, sem_a0, sem_b0)

    def pair(cj, carry):
        c0 = 2 * cj
        start(c0 + 1, buf_a1, buf_b1, sem_a1, sem_b1)
        finish(c0, buf_a0, buf_b0, sem_a0, sem_b0)
        start(c0 + 2, buf_a0, buf_b0, sem_a0, sem_b0)
        finish(c0 + 1, buf_a1, buf_b1, sem_a1, sem_b1)
        return carry

    lax.fori_loop(0, (n_chunks - 1) // 2, pair, 0)
    finish(n_chunks - 1, buf_a0, buf_b0, sem_a0, sem_b0)


def _gather_add(p_src, p_dst, src3, dst3):
    n, d2 = p_src.shape
    nw, nch, cb = src3.shape
    e = nw * nch * cb
    mesh = plsc.VectorSubcoreMesh(core_axis_name="c", subcore_axis_name="s")
    kfn = pl.kernel(
        functools.partial(_gather_body, e, d2),
        out_type=jax.ShapeDtypeStruct((e, d2), jnp.float32),
        mesh=mesh,
        scratch_types=[
            pltpu.VMEM((nch, cb), jnp.int32),
            pltpu.VMEM((nch, cb), jnp.int32),
            pltpu.VMEM((_CB, d2), jnp.float32),
            pltpu.VMEM((_CB, d2), jnp.float32),
            pltpu.VMEM((_CB, d2), jnp.float32),
            pltpu.VMEM((_CB, d2), jnp.float32),
            pltpu.SemaphoreType.DMA,
            pltpu.SemaphoreType.DMA,
            pltpu.SemaphoreType.DMA,
            pltpu.SemaphoreType.DMA,
        ],
    )
    return kfn(p_src, p_dst, src3, dst3)


def _edge_body(d, g_ref, ea_ref, we_ref, b_ref, out_ref):
    x = (g_ref[...]
         + jnp.dot(ea_ref[...], we_ref[...], preferred_element_type=jnp.float32)
         + b_ref[...])
    xg = x[:, :d]
    xm = x[:, d:]
    gate = 1.0 / (1.0 + jnp.exp(-xg))
    msg = jnp.maximum(xm, 0.0) + jnp.log(1.0 + jnp.exp(-jnp.abs(xm)))
    out_ref[...] = gate * msg


def _edge_mlp(g, edge_attr, w_edge, b_cat):
    e, d2 = g.shape
    de = edge_attr.shape[1]
    d = d2 // 2
    be = 2000
    return pl.pallas_call(
        functools.partial(_edge_body, d),
        grid=(e // be,),
        in_specs=[
            pl.BlockSpec((be, d2), lambda i: (i, 0)),
            pl.BlockSpec((be, de), lambda i: (i, 0)),
            pl.BlockSpec((de, d2), lambda i: (0, 0)),
            pl.BlockSpec((1, d2), lambda i: (0, 0)),
        ],
        out_specs=pl.BlockSpec((be, d), lambda i: (i, 0)),
        out_shape=jax.ShapeDtypeStruct((e, d), jnp.float32),
    )(g, edge_attr, w_edge, b_cat)


_RB = 128


def _scatter_body(e, n_pad, d, gm_hbm, dst3_hbm, s_out,
                  idx2, gm_buf, row_buf, s_sh):
    c = lax.axis_index("c")
    s = lax.axis_index("s")
    wid = c * _NS + s
    ept = e // (_NC * _NS)
    base = wid * ept
    rpt = n_pad // _NS
    r0 = s * rpt

    zv = jnp.zeros((_L,), jnp.float32)

    def zrow(j, cc):
        for k in range(d // _L):
            row_buf[j, pl.ds(k * _L, _L)] = zv
        return cc

    lax.fori_loop(0, _RB, zrow, 0)
    pltpu.sync_copy(dst3_hbm.at[wid], idx2)
    for j in range(rpt // _RB):
        pltpu.sync_copy(row_buf, s_sh.at[pl.ds(r0 + j * _RB, _RB)])
    plsc.subcore_barrier()

    def chunk(ci, carry):
        off = base + ci * _CB
        pltpu.sync_copy(gm_hbm.at[pl.ds(off, _CB)], gm_buf)
        pltpu.sync_copy(gm_buf, s_sh.at[idx2.at[ci]], add=True)
        return carry

    lax.fori_loop(0, ept // _CB, chunk, 0)
    plsc.subcore_barrier()
    for j in range(rpt // _RB):
        pltpu.sync_copy(s_sh.at[pl.ds(r0 + j * _RB, _RB)], row_buf)
        pltpu.sync_copy(row_buf, s_out.at[c, pl.ds(r0 + j * _RB, _RB)])


def _count_body(e, n_pad, dst_hbm, c_out, idx1d, cnt1d):
    c = lax.axis_index("c")
    s = lax.axis_index("s")
    wid = c * _NS + s
    ept = e // (_NC * _NS)
    base = wid * ept

    zv = jnp.zeros((_L,), jnp.float32)
    ones_v = jnp.full((_L,), 1.0, jnp.float32)

    def zcnt(j, cc):
        cnt1d[pl.ds(j * _L, _L)] = zv
        return cc

    lax.fori_loop(0, n_pad // _L, zcnt, 0)
    pltpu.sync_copy(dst_hbm.at[pl.ds(base, ept)], idx1d)

    def step(k, cc):
        iv = idx1d[pl.ds(k * _L, _L)]
        plsc.addupdate_scatter(cnt1d, [iv], ones_v)
        return cc

    lax.fori_loop(0, ept // _L, step, 0)
    pltpu.sync_copy(cnt1d, c_out.at[wid, 0])


def _scatter_mean_parts(gm, dst, dst3, n):
    e, d = gm.shape
    blk = _NS * _RB
    n_pad = ((n + blk - 1) // blk) * blk
    nch = dst3.shape[1]
    mesh = plsc.VectorSubcoreMesh(core_axis_name="c", subcore_axis_name="s")
    kfn = pl.kernel(
        functools.partial(_scatter_body, e, n_pad, d),
        out_type=jax.ShapeDtypeStruct((_NC, n_pad, d), jnp.float32),
        mesh=mesh,
        scratch_types=[
            pltpu.VMEM((nch, _CB), jnp.int32),
            pltpu.VMEM((_CB, d), jnp.float32),
            pltpu.VMEM((_RB, d), jnp.float32),
            pltpu.VMEM_SHARED((n_pad, d), jnp.float32),
        ],
    )
    s_parts = kfn(gm, dst3)
    cfn = pl.kernel(
        functools.partial(_count_body, e, n_pad),
        out_type=jax.ShapeDtypeStruct((_NC * _NS, 1, n_pad), jnp.float32),
        mesh=plsc.VectorSubcoreMesh(core_axis_name="c", subcore_axis_name="s"),
        compiler_params=pltpu.CompilerParams(
            use_tc_tiling_on_sc=False, needs_layout_passes=False),
        scratch_types=[
            pltpu.VMEM((e // (_NC * _NS),), jnp.int32),
            pltpu.VMEM((n_pad,), jnp.float32),
        ],
    )
    c_parts = cfn(dst)
    return s_parts, c_parts


def _fin_body(bn, h_ref, s_ref, c_ref, out_ref):
    i = pl.program_id(0)
    ssum = s_ref[0] + s_ref[1]
    cblk = c_ref[:, :, pl.ds(i * bn, bn)]
    cnt = jnp.sum(cblk, axis=(0, 1))[:, None]
    out_ref[...] = h_ref[...] + ssum / jnp.maximum(cnt, 1.0)


def _finalize(h, s_parts, c_parts):
    n, d = h.shape
    n_pad = c_parts.shape[2]
    h_pad = jnp.pad(h, ((0, n_pad - n), (0, 0)))
    bn = 2048
    out = pl.pallas_call(
        functools.partial(_fin_body, bn),
        grid=(n_pad // bn,),
        in_specs=[
            pl.BlockSpec((bn, d), lambda i: (i, 0)),
            pl.BlockSpec((_NC, bn, d), lambda i: (0, i, 0)),
            pl.BlockSpec((_NC * _NS, 1, n_pad), lambda i: (0, 0, 0)),
        ],
        out_specs=pl.BlockSpec((bn, d), lambda i: (i, 0)),
        out_shape=jax.ShapeDtypeStruct((n_pad, d), jnp.float32),
    )(h_pad, s_parts, c_parts)
    return out[:n]


def kernel(h, edge_index, edge_attr, W_e, b_e, W_n, b_n):
    n, d = h.shape
    e = edge_index.shape[1]
    nw = _NC * _NS
    src = edge_index[0]
    dst = edge_index[1]
    src3 = src.reshape(nw, e // (nw * _CB), _CB)
    dst3 = dst.reshape(nw, e // (nw * _CB), _CB)
    w_src = jnp.concatenate([W_e[:d], W_n[:d]], axis=1)
    w_dst = jnp.concatenate([W_e[d:2 * d], W_n[d:2 * d]], axis=1)
    w_edge = jnp.concatenate([W_e[2 * d:], W_n[2 * d:]], axis=1)
    b_cat = jnp.concatenate([b_e, b_n])[None, :]

    p_src, p_dst = _project(h, w_src, w_dst)
    g = _gather_add(p_src, p_dst, src3, dst3)
    gm = _edge_mlp(g, edge_attr, w_edge, b_cat)
    s_parts, c_parts = _scatter_mean_parts(gm, dst, dst3, n)
    return _finalize(h, s_parts, c_parts)
